# baseline (speedup 1.0000x reference)
import functools

import jax
import jax.numpy as jnp
from jax import lax
from jax.experimental import pallas as pl
from jax.experimental.pallas import tpu as pltpu
from jax.experimental.pallas import tpu_sc as plsc

VOCAB = 100000
DIM = 64
BATCH = 4096
CTX = 20

NC = 2
NS = 16
NW = NC * NS
BPW = BATCH // NW
HALF = CTX // 2
LANES = 16


def _sc_mean_body(idx_hbm, emb_hbm, out_hbm, idx_v, rows_v, acc_v, sem):
    wid = lax.axis_index("s") * NC + lax.axis_index("c")
    pltpu.sync_copy(idx_hbm.at[wid], idx_v)

    for half in range(CTX // HALF):
        copies = [
            pltpu.make_async_copy(
                emb_hbm.at[idx_v.at[half * HALF + j]], rows_v.at[j], sem)
            for j in range(HALF)
        ]
        for c in copies:
            c.start()
        for c in copies:
            c.wait()

        def body(r, carry):
            for k in range(DIM // LANES):
                s = pl.ds(k * LANES, LANES)
                acc = rows_v[0, r, s]
                for j in range(1, HALF):
                    acc = acc + rows_v[j, r, s]
                if half == 0:
                    acc_v[r, s] = acc
                else:
                    acc_v[r, s] = (acc_v[r, s] + acc) * (1.0 / CTX)
            return carry

        lax.fori_loop(0, BPW, body, 0, unroll=False)

    pltpu.sync_copy(acc_v, out_hbm.at[pl.ds(wid * BPW, BPW)])


@functools.cache
def _sc_mean():
    return pl.kernel(
        _sc_mean_body,
        out_type=jax.ShapeDtypeStruct((BATCH, DIM), jnp.float32),
        mesh=plsc.VectorSubcoreMesh(
            core_axis_name="c", subcore_axis_name="s",
            num_cores=NC, num_subcores=NS),
        scratch_types=[
            pltpu.VMEM((CTX, BPW), jnp.int32),
            pltpu.VMEM((HALF, BPW, DIM), jnp.float32),
            pltpu.VMEM((BPW, DIM), jnp.float32),
            pltpu.SemaphoreType.DMA,
        ],
        compiler_params=pltpu.CompilerParams(use_tc_tiling_on_sc=False),
    )


BT = 32
NB = BATCH // BT


def _mm_body(x_ref, w_ref, b_ref, o_ref):
    o_ref[...] = (
        jnp.dot(x_ref[...], w_ref[...], preferred_element_type=jnp.float32)
        + b_ref[...]
    )


def _project(mean, W, b2):
    return pl.pallas_call(
        _mm_body,
        grid=(NB,),
        in_specs=[
            pl.BlockSpec((BT, DIM), lambda i: (i, 0)),
            pl.BlockSpec((DIM, VOCAB), lambda i: (0, 0)),
            pl.BlockSpec((1, VOCAB), lambda i: (0, 0)),
        ],
        out_specs=pl.BlockSpec((BT, VOCAB), lambda i: (i, 0)),
        out_shape=jax.ShapeDtypeStruct((BATCH, VOCAB), jnp.float32),
    )(mean, W, b2)


def kernel(context_indices, emb, W, b):
    idx = context_indices.astype(jnp.int32).reshape(NW, BPW, CTX).swapaxes(1, 2)
    mean = _sc_mean()(idx, emb)
    return _project(mean, W, b.reshape(1, VOCAB))


# speedup vs baseline: 1.0000x; 1.0000x over previous
import functools

import jax
import jax.numpy as jnp
from jax import lax
from jax.experimental import pallas as pl
from jax.experimental.pallas import tpu as pltpu
from jax.experimental.pallas import tpu_sc as plsc

VOCAB = 100000
DIM = 64
DIMP = 128
BATCH = 4096
CTX = 20
CTXP = 24

NC = 2
NS = 16
NW = NC * NS
BPW = BATCH // NW
PASS = 5
NPASS = CTX // PASS
LANES = 16


def _sc_mean_body(idx_hbm, emb_hbm, out_hbm, idx_v, rows_v, acc_v, sem):
    wid = lax.axis_index("s") * NC + lax.axis_index("c")
    pltpu.sync_copy(idx_hbm.at[wid], idx_v)

    for p in range(NPASS):
        copies = [
            pltpu.make_async_copy(
                emb_hbm.at[idx_v.at[p * PASS + j]], rows_v.at[j], sem)
            for j in range(PASS)
        ]
        for c in copies:
            c.start()
        for c in copies:
            c.wait()

        def body(r, carry):
            for k in range(DIM // LANES):
                s = pl.ds(k * LANES, LANES)
                acc = rows_v[0, r, s]
                for j in range(1, PASS):
                    acc = acc + rows_v[j, r, s]
                if p == 0:
                    acc_v[r, s] = acc
                elif p < NPASS - 1:
                    acc_v[r, s] = acc_v[r, s] + acc
                else:
                    acc_v[r, s] = (acc_v[r, s] + acc) * (1.0 / CTX)
            return carry

        lax.fori_loop(0, BPW, body, 0, unroll=False)

    pltpu.sync_copy(acc_v, out_hbm.at[pl.ds(wid * BPW, BPW)])


@functools.cache
def _sc_mean():
    return pl.kernel(
        _sc_mean_body,
        out_type=jax.ShapeDtypeStruct((BATCH, DIMP), jnp.float32),
        mesh=plsc.VectorSubcoreMesh(
            core_axis_name="c", subcore_axis_name="s",
            num_cores=NC, num_subcores=NS),
        scratch_types=[
            pltpu.VMEM((CTXP, BPW), jnp.int32),
            pltpu.VMEM((PASS, BPW, DIMP), jnp.float32),
            pltpu.VMEM((BPW, DIMP), jnp.float32),
            pltpu.SemaphoreType.DMA,
        ],
    )


BT = 32
NB = BATCH // BT


def _mm_body(x_ref, w_ref, b_ref, o_ref):
    o_ref[...] = (
        jnp.dot(x_ref[...], w_ref[...], preferred_element_type=jnp.float32)
        + b_ref[...]
    )


def _project(mean, W, b2):
    return pl.pallas_call(
        _mm_body,
        grid=(NB,),
        in_specs=[
            pl.BlockSpec((BT, DIM), lambda i: (i, 0)),
            pl.BlockSpec((DIM, VOCAB), lambda i: (0, 0)),
            pl.BlockSpec((1, VOCAB), lambda i: (0, 0)),
        ],
        out_specs=pl.BlockSpec((BT, VOCAB), lambda i: (i, 0)),
        out_shape=jax.ShapeDtypeStruct((BATCH, VOCAB), jnp.float32),
    )(mean, W, b2)


def kernel(context_indices, emb, W, b):
    emb_pad = jnp.pad(emb, ((0, 0), (0, DIMP - DIM)))
    idx = context_indices.astype(jnp.int32).reshape(NW, BPW, CTX).swapaxes(1, 2)
    idx = jnp.pad(idx, ((0, 0), (0, CTXP - CTX), (0, 0)))
    mean = _sc_mean()(idx, emb_pad)[:, :DIM]
    return _project(mean, W, b.reshape(1, VOCAB))


# speedup vs baseline: 3.1502x; 3.1501x over previous
import functools

import jax
import jax.numpy as jnp
from jax import lax
from jax.experimental import pallas as pl
from jax.experimental.pallas import tpu as pltpu
from jax.experimental.pallas import tpu_sc as plsc

VOCAB = 100000
DIM = 64
DIMP = 128
BATCH = 4096
CTX = 20
CTXP = 24

NC = 2
NS = 16
NW = NC * NS
BPW = BATCH // NW
PASS = 5
NPASS = CTX // PASS
LANES = 16


def _sc_mean_body(idx_hbm, emb_hbm, out_hbm, idx_v, rows_v, acc_v, sem):
    wid = lax.axis_index("s") * NC + lax.axis_index("c")
    pltpu.sync_copy(idx_hbm.at[wid], idx_v)

    for p in range(NPASS):
        copies = [
            pltpu.make_async_copy(
                emb_hbm.at[idx_v.at[p * PASS + j]], rows_v.at[j], sem)
            for j in range(PASS)
        ]
        for c in copies:
            c.start()
        for c in copies:
            c.wait()

        def body(r, carry):
            for k in range(DIM // LANES):
                s = pl.ds(k * LANES, LANES)
                acc = rows_v[0, r, s]
                for j in range(1, PASS):
                    acc = acc + rows_v[j, r, s]
                if p == 0:
                    acc_v[r, s] = acc
                elif p < NPASS - 1:
                    acc_v[r, s] = acc_v[r, s] + acc
                else:
                    acc_v[r, s] = (acc_v[r, s] + acc) * (1.0 / CTX)
            return carry

        lax.fori_loop(0, BPW, body, 0, unroll=False)

    pltpu.sync_copy(acc_v, out_hbm.at[pl.ds(wid * BPW, BPW)])


@functools.cache
def _sc_mean():
    return pl.kernel(
        _sc_mean_body,
        out_type=jax.ShapeDtypeStruct((BATCH, DIMP), jnp.float32),
        mesh=plsc.VectorSubcoreMesh(
            core_axis_name="c", subcore_axis_name="s",
            num_cores=NC, num_subcores=NS),
        scratch_types=[
            pltpu.VMEM((CTXP, BPW), jnp.int32),
            pltpu.VMEM((PASS, BPW, DIMP), jnp.float32),
            pltpu.VMEM((BPW, DIMP), jnp.float32),
            pltpu.SemaphoreType.DMA,
        ],
    )


VS = 512
NV = (VOCAB + VS - 1) // VS


def _mm_body(w_ref, x_ref, b_ref, o_ref):
    o_ref[...] = (
        lax.dot_general(w_ref[...], x_ref[...], (((0,), (0,)), ((), ())),
                        preferred_element_type=jnp.float32)
        + b_ref[...]
    )


def _project(W, meanT, b2):
    return pl.pallas_call(
        _mm_body,
        grid=(NV,),
        in_specs=[
            pl.BlockSpec((DIM, VS), lambda i: (0, i)),
            pl.BlockSpec((DIM, BATCH), lambda i: (0, 0)),
            pl.BlockSpec((VS, 1), lambda i: (i, 0)),
        ],
        out_specs=pl.BlockSpec((VS, BATCH), lambda i: (i, 0)),
        out_shape=jax.ShapeDtypeStruct((VOCAB, BATCH), jnp.float32),
    )(W, meanT, b2)


def kernel(context_indices, emb, W, b):
    emb_pad = jnp.pad(emb, ((0, 0), (0, DIMP - DIM)))
    idx = context_indices.astype(jnp.int32).reshape(NW, BPW, CTX).swapaxes(1, 2)
    idx = jnp.pad(idx, ((0, 0), (0, CTXP - CTX), (0, 0)))
    meanT = _sc_mean()(idx, emb_pad).T[:DIM]
    return _project(W, meanT, b.reshape(VOCAB, 1)).T


# speedup vs baseline: 3.1904x; 1.0128x over previous
import functools

import jax
import jax.numpy as jnp
from jax import lax
from jax.experimental import pallas as pl
from jax.experimental.pallas import tpu as pltpu
from jax.experimental.pallas import tpu_sc as plsc

VOCAB = 100000
DIM = 64
DIMP = 128
BATCH = 4096
CTX = 20
CTXP = 24

NC = 2
NS = 16
NW = NC * NS
BPW = BATCH // NW
PASS = 5
NPASS = CTX // PASS
LANES = 16


def _sc_mean_body(idx_hbm, emb_hbm, out_hbm, idx_v, rows_v, acc_v, sem):
    wid = lax.axis_index("s") * NC + lax.axis_index("c")
    pltpu.sync_copy(idx_hbm.at[wid], idx_v)

    for p in range(NPASS):
        copies = [
            pltpu.make_async_copy(
                emb_hbm.at[idx_v.at[p * PASS + j]], rows_v.at[j], sem)
            for j in range(PASS)
        ]
        for c in copies:
            c.start()
        for c in copies:
            c.wait()

        def body(r, carry):
            for k in range(DIM // LANES):
                s = pl.ds(k * LANES, LANES)
                acc = rows_v[0, r, s]
                for j in range(1, PASS):
                    acc = acc + rows_v[j, r, s]
                if p == 0:
                    acc_v[r, s] = acc
                elif p < NPASS - 1:
                    acc_v[r, s] = acc_v[r, s] + acc
                else:
                    acc_v[r, s] = (acc_v[r, s] + acc) * (1.0 / CTX)
            return carry

        lax.fori_loop(0, BPW, body, 0, unroll=False)

    pltpu.sync_copy(acc_v, out_hbm.at[pl.ds(wid * BPW, BPW)])


@functools.cache
def _sc_mean():
    return pl.kernel(
        _sc_mean_body,
        out_type=jax.ShapeDtypeStruct((BATCH, DIMP), jnp.float32),
        mesh=plsc.VectorSubcoreMesh(
            core_axis_name="c", subcore_axis_name="s",
            num_cores=NC, num_subcores=NS),
        scratch_types=[
            pltpu.VMEM((CTXP, BPW), jnp.int32),
            pltpu.VMEM((PASS, BPW, DIMP), jnp.float32),
            pltpu.VMEM((BPW, DIMP), jnp.float32),
            pltpu.SemaphoreType.DMA,
        ],
    )


VS = 1024
NV = (VOCAB + VS - 1) // VS


def _mm_body(w_ref, x_ref, b_ref, o_ref):
    o_ref[...] = (
        lax.dot_general(w_ref[...], x_ref[...], (((0,), (0,)), ((), ())),
                        preferred_element_type=jnp.float32)
        + b_ref[...]
    )


def _project(W, meanT, b2):
    return pl.pallas_call(
        _mm_body,
        grid=(NV,),
        in_specs=[
            pl.BlockSpec((DIM, VS), lambda i: (0, i)),
            pl.BlockSpec((DIM, BATCH), lambda i: (0, 0)),
            pl.BlockSpec((VS, 1), lambda i: (i, 0)),
        ],
        out_specs=pl.BlockSpec((VS, BATCH), lambda i: (i, 0)),
        out_shape=jax.ShapeDtypeStruct((VOCAB, BATCH), jnp.float32),
    )(W, meanT, b2)


def kernel(context_indices, emb, W, b):
    emb_pad = jnp.pad(emb, ((0, 0), (0, DIMP - DIM)))
    idx = context_indices.astype(jnp.int32).reshape(NW, BPW, CTX).swapaxes(1, 2)
    idx = jnp.pad(idx, ((0, 0), (0, CTXP - CTX), (0, 0)))
    meanT = _sc_mean()(idx, emb_pad).T[:DIM]
    return _project(W, meanT, b.reshape(VOCAB, 1)).T


# speedup vs baseline: 3.2417x; 1.0161x over previous
import functools

import jax
import jax.numpy as jnp
from jax import lax
from jax.experimental import pallas as pl
from jax.experimental.pallas import tpu as pltpu
from jax.experimental.pallas import tpu_sc as plsc

VOCAB = 100000
DIM = 64
DIMP = 128
BATCH = 4096
CTX = 20
CTXP = 24

NC = 2
NS = 16
NW = NC * NS
BPW = BATCH // NW
PASS = 5
NPASS = CTX // PASS
LANES = 16


def _sc_mean_body(idx_hbm, emb_hbm, out_hbm, idx_v, rows_v, acc_v, sem):
    wid = lax.axis_index("s") * NC + lax.axis_index("c")
    pltpu.sync_copy(idx_hbm.at[wid], idx_v)

    for p in range(NPASS):
        copies = [
            pltpu.make_async_copy(
                emb_hbm.at[idx_v.at[p * PASS + j]], rows_v.at[j], sem)
            for j in range(PASS)
        ]
        for c in copies:
            c.start()
        for c in copies:
            c.wait()

        def body(r, carry):
            for k in range(DIM // LANES):
                s = pl.ds(k * LANES, LANES)
                acc = rows_v[0, r, s]
                for j in range(1, PASS):
                    acc = acc + rows_v[j, r, s]
                if p == 0:
                    acc_v[r, s] = acc
                elif p < NPASS - 1:
                    acc_v[r, s] = acc_v[r, s] + acc
                else:
                    acc_v[r, s] = (acc_v[r, s] + acc) * (1.0 / CTX)
            return carry

        lax.fori_loop(0, BPW, body, 0, unroll=False)

    pltpu.sync_copy(acc_v, out_hbm.at[pl.ds(wid * BPW, BPW)])


@functools.cache
def _sc_mean():
    return pl.kernel(
        _sc_mean_body,
        out_type=jax.ShapeDtypeStruct((BATCH, DIMP), jnp.float32),
        mesh=plsc.VectorSubcoreMesh(
            core_axis_name="c", subcore_axis_name="s",
            num_cores=NC, num_subcores=NS),
        scratch_types=[
            pltpu.VMEM((CTXP, BPW), jnp.int32),
            pltpu.VMEM((PASS, BPW, DIMP), jnp.float32),
            pltpu.VMEM((BPW, DIMP), jnp.float32),
            pltpu.SemaphoreType.DMA,
        ],
    )


VS = 1024
NV = (VOCAB + VS - 1) // VS


def _mm_body(w_ref, x_ref, b_ref, o_ref):
    o_ref[...] = (
        lax.dot_general(w_ref[...], x_ref[...], (((0,), (1,)), ((), ())),
                        preferred_element_type=jnp.float32)
        + b_ref[...]
    )


def _project(W, meanT, b2):
    return pl.pallas_call(
        _mm_body,
        grid=(NV,),
        in_specs=[
            pl.BlockSpec((DIM, VS), lambda i: (0, i)),
            pl.BlockSpec((BATCH, DIM), lambda i: (0, 0)),
            pl.BlockSpec((VS, 1), lambda i: (i, 0)),
        ],
        out_specs=pl.BlockSpec((VS, BATCH), lambda i: (i, 0)),
        out_shape=jax.ShapeDtypeStruct((VOCAB, BATCH), jnp.float32),
    )(W, meanT, b2)


def kernel(context_indices, emb, W, b):
    emb_pad = jnp.pad(emb, ((0, 0), (0, DIMP - DIM)))
    idx = context_indices.astype(jnp.int32).reshape(NW, BPW, CTX).swapaxes(1, 2)
    idx = jnp.pad(idx, ((0, 0), (0, CTXP - CTX), (0, 0)))
    mean = _sc_mean()(idx, emb_pad)[:, :DIM]
    return _project(W, mean, b.reshape(VOCAB, 1)).T
